# baseline (device time: 142476 ns/iter reference)
import jax
import jax.numpy as jnp
from jax import lax
from jax.experimental import pallas as pl
from jax.experimental.pallas import tpu as pltpu

T = 512
D = 1024
V_SHARD = 8192
NCHUNK = 8
CC = V_SHARD // NCHUNK


def kernel(x, W):
    def body(
        x_ref, w_ref, out_ref,
        wbuf, send_buf, recv_buf, stage, sum_send, sum_recv,
        w_sems, send_sems, recv_sems, out_sems, sum_sems,
    ):
        my_x = lax.axis_index("x")
        my_y = lax.axis_index("y")
        my_z = lax.axis_index("z")
        peer = (my_x, my_y, 1 - my_z)

        barrier_sem = pltpu.get_barrier_semaphore()
        pl.semaphore_signal(
            barrier_sem, inc=1, device_id=peer,
            device_id_type=pl.DeviceIdType.MESH,
        )
        pl.semaphore_wait(barrier_sem, 1)

        def start_w(c):
            cp = pltpu.make_async_copy(
                w_ref.at[:, pl.ds(c * CC, CC)],
                wbuf.at[c % 2],
                w_sems.at[c % 2],
            )
            cp.start()
            return cp

        w_copies = [start_w(0), start_w(1)]

        xb = x_ref[...].astype(jnp.bfloat16)
        my_off = my_z * V_SHARD
        peer_off = (1 - my_z) * V_SHARD

        rdmas = []
        s_mine = jnp.zeros((T, 1), jnp.float32)
        for c in range(NCHUNK):
            w_copies[c].wait()
            wb = wbuf[c % 2].astype(jnp.bfloat16)
            e = jnp.exp(jnp.dot(xb, wb, preferred_element_type=jnp.float32))
            s_mine = s_mine + jnp.sum(e, axis=1, keepdims=True)
            send_buf[c] = e.astype(jnp.bfloat16)
            if c + 2 < NCHUNK:
                w_copies.append(start_w(c + 2))
            rdma = pltpu.make_async_remote_copy(
                src_ref=send_buf.at[c],
                dst_ref=recv_buf.at[c],
                send_sem=send_sems.at[c],
                recv_sem=recv_sems.at[c],
                device_id=peer,
                device_id_type=pl.DeviceIdType.MESH,
            )
            rdma.start()
            rdmas.append(rdma)

        sum_send[...] = s_mine
        sum_rdma = pltpu.make_async_remote_copy(
            src_ref=sum_send,
            dst_ref=sum_recv,
            send_sem=sum_sems.at[0],
            recv_sem=sum_sems.at[1],
            device_id=peer,
            device_id_type=pl.DeviceIdType.MESH,
        )
        sum_rdma.start()
        sum_rdma.wait_recv()
        inv = 1.0 / (s_mine + sum_recv[...])

        CS = CC // 2
        out_copies = []
        def store_piece(idx, val, col_off):
            slot = idx % 2
            if idx >= 2:
                out_copies[idx - 2].wait()
            stage[slot] = val
            cp = pltpu.make_async_copy(
                stage.at[slot],
                out_ref.at[:, pl.ds(col_off, CS)],
                out_sems.at[slot],
            )
            cp.start()
            out_copies.append(cp)

        idx = 0
        for c in range(NCHUNK):
            for h in range(2):
                lo, hi = h * CS, (h + 1) * CS
                store_piece(
                    idx,
                    send_buf[c][:, lo:hi].astype(jnp.float32) * inv,
                    my_off + c * CC + h * CS,
                )
                idx += 1
        for c in range(NCHUNK):
            rdmas[c].wait_recv()
            for h in range(2):
                lo, hi = h * CS, (h + 1) * CS
                store_piece(
                    idx,
                    recv_buf[c][:, lo:hi].astype(jnp.float32) * inv,
                    peer_off + c * CC + h * CS,
                )
                idx += 1
        out_copies[-2].wait()
        out_copies[-1].wait()

        for c in range(NCHUNK):
            rdmas[c].wait_send()
        sum_rdma.wait_send()

    return pl.pallas_call(
        body,
        out_shape=jax.ShapeDtypeStruct((T, 2 * V_SHARD), jnp.float32),
        in_specs=[
            pl.BlockSpec(memory_space=pltpu.VMEM),
            pl.BlockSpec(memory_space=pltpu.HBM),
        ],
        out_specs=pl.BlockSpec(memory_space=pltpu.HBM),
        scratch_shapes=[
            pltpu.VMEM((2, D, CC), jnp.float32),
            pltpu.VMEM((NCHUNK, T, CC), jnp.bfloat16),
            pltpu.VMEM((NCHUNK, T, CC), jnp.bfloat16),
            pltpu.VMEM((2, T, CC // 2), jnp.float32),
            pltpu.VMEM((T, 1), jnp.float32),
            pltpu.VMEM((T, 1), jnp.float32),
            pltpu.SemaphoreType.DMA((2,)),
            pltpu.SemaphoreType.DMA((NCHUNK,)),
            pltpu.SemaphoreType.DMA((NCHUNK,)),
            pltpu.SemaphoreType.DMA((2,)),
            pltpu.SemaphoreType.DMA((2,)),
        ],
        compiler_params=pltpu.CompilerParams(collective_id=0),
    )(x, W)


# device time: 126357 ns/iter; 1.1276x vs baseline; 1.1276x over previous
import jax
import jax.numpy as jnp
from jax import lax
from jax.experimental import pallas as pl
from jax.experimental.pallas import tpu as pltpu

T = 512
D = 1024
V_SHARD = 8192
NCHUNK = 8
CC = V_SHARD // NCHUNK


def kernel(x, W):
    def body(
        x_ref, w_ref, out_ref,
        wbuf, send_buf, recv_buf, stage, sum_send, sum_recv,
        w_sems, send_sems, recv_sems, out_sems, sum_sems,
    ):
        my_x = lax.axis_index("x")
        my_y = lax.axis_index("y")
        my_z = lax.axis_index("z")
        peer = (my_x, my_y, 1 - my_z)

        barrier_sem = pltpu.get_barrier_semaphore()
        pl.semaphore_signal(
            barrier_sem, inc=1, device_id=peer,
            device_id_type=pl.DeviceIdType.MESH,
        )
        pl.semaphore_wait(barrier_sem, 1)

        def start_w(c):
            cp = pltpu.make_async_copy(
                w_ref.at[:, pl.ds(c * CC, CC)],
                wbuf.at[c % 2],
                w_sems.at[c % 2],
            )
            cp.start()
            return cp

        w_copies = [start_w(0), start_w(1)]

        xb = x_ref[...].astype(jnp.bfloat16)
        my_off = my_z * V_SHARD
        peer_off = (1 - my_z) * V_SHARD

        rdmas = []
        s_mine = jnp.zeros((T, 1), jnp.float32)
        for c in range(NCHUNK):
            w_copies[c].wait()
            wb = wbuf[c % 2].astype(jnp.bfloat16)
            e = jnp.exp(jnp.dot(xb, wb, preferred_element_type=jnp.float32))
            s_mine = s_mine + jnp.sum(e, axis=1, keepdims=True)
            send_buf[c] = e.astype(jnp.bfloat16)
            if c + 2 < NCHUNK:
                w_copies.append(start_w(c + 2))
            rdma = pltpu.make_async_remote_copy(
                src_ref=send_buf.at[c],
                dst_ref=recv_buf.at[c],
                send_sem=send_sems.at[c],
                recv_sem=recv_sems.at[c],
                device_id=peer,
                device_id_type=pl.DeviceIdType.MESH,
            )
            if c < NCHUNK // 2:
                rdma.start()
            rdmas.append(rdma)

        sum_send[...] = s_mine
        sum_rdma = pltpu.make_async_remote_copy(
            src_ref=sum_send,
            dst_ref=sum_recv,
            send_sem=sum_sems.at[0],
            recv_sem=sum_sems.at[1],
            device_id=peer,
            device_id_type=pl.DeviceIdType.MESH,
        )
        sum_rdma.start()
        for c in range(NCHUNK // 2, NCHUNK):
            rdmas[c].start()
        sum_rdma.wait_recv()
        inv = 1.0 / (s_mine + sum_recv[...])

        CS = CC // 2
        out_copies = []
        def store_piece(idx, val, col_off):
            slot = idx % 2
            if idx >= 2:
                out_copies[idx - 2].wait()
            stage[slot] = val
            cp = pltpu.make_async_copy(
                stage.at[slot],
                out_ref.at[:, pl.ds(col_off, CS)],
                out_sems.at[slot],
            )
            cp.start()
            out_copies.append(cp)

        idx = 0
        for c in range(NCHUNK):
            for h in range(2):
                lo, hi = h * CS, (h + 1) * CS
                store_piece(
                    idx,
                    send_buf[c][:, lo:hi].astype(jnp.float32) * inv,
                    my_off + c * CC + h * CS,
                )
                idx += 1
        for c in range(NCHUNK):
            rdmas[c].wait_recv()
            for h in range(2):
                lo, hi = h * CS, (h + 1) * CS
                store_piece(
                    idx,
                    recv_buf[c][:, lo:hi].astype(jnp.float32) * inv,
                    peer_off + c * CC + h * CS,
                )
                idx += 1
        out_copies[-2].wait()
        out_copies[-1].wait()

        for c in range(NCHUNK):
            rdmas[c].wait_send()
        sum_rdma.wait_send()

    return pl.pallas_call(
        body,
        out_shape=jax.ShapeDtypeStruct((T, 2 * V_SHARD), jnp.float32),
        in_specs=[
            pl.BlockSpec(memory_space=pltpu.VMEM),
            pl.BlockSpec(memory_space=pltpu.HBM),
        ],
        out_specs=pl.BlockSpec(memory_space=pltpu.HBM),
        scratch_shapes=[
            pltpu.VMEM((2, D, CC), jnp.float32),
            pltpu.VMEM((NCHUNK, T, CC), jnp.bfloat16),
            pltpu.VMEM((NCHUNK, T, CC), jnp.bfloat16),
            pltpu.VMEM((2, T, CC // 2), jnp.float32),
            pltpu.VMEM((T, 1), jnp.float32),
            pltpu.VMEM((T, 1), jnp.float32),
            pltpu.SemaphoreType.DMA((2,)),
            pltpu.SemaphoreType.DMA((NCHUNK,)),
            pltpu.SemaphoreType.DMA((NCHUNK,)),
            pltpu.SemaphoreType.DMA((2,)),
            pltpu.SemaphoreType.DMA((2,)),
        ],
        compiler_params=pltpu.CompilerParams(collective_id=0),
    )(x, W)


# device time: 124518 ns/iter; 1.1442x vs baseline; 1.0148x over previous
import jax
import jax.numpy as jnp
from jax import lax
from jax.experimental import pallas as pl
from jax.experimental.pallas import tpu as pltpu

T = 512
D = 1024
V_SHARD = 8192
NCHUNK = 16
CC = V_SHARD // NCHUNK


def kernel(x, W):
    def body(
        x_ref, w_ref, out_ref,
        wbuf, send_buf, recv_buf, stage, sum_send, sum_recv,
        w_sems, send_sems, recv_sems, out_sems, sum_sems,
    ):
        my_x = lax.axis_index("x")
        my_y = lax.axis_index("y")
        my_z = lax.axis_index("z")
        peer = (my_x, my_y, 1 - my_z)

        barrier_sem = pltpu.get_barrier_semaphore()
        pl.semaphore_signal(
            barrier_sem, inc=1, device_id=peer,
            device_id_type=pl.DeviceIdType.MESH,
        )
        pl.semaphore_wait(barrier_sem, 1)

        def start_w(c):
            cp = pltpu.make_async_copy(
                w_ref.at[:, pl.ds(c * CC, CC)],
                wbuf.at[c % 2],
                w_sems.at[c % 2],
            )
            cp.start()
            return cp

        w_copies = [start_w(0), start_w(1)]

        xb = x_ref[...].astype(jnp.bfloat16)
        my_off = my_z * V_SHARD
        peer_off = (1 - my_z) * V_SHARD

        rdmas = []
        s_mine = jnp.zeros((T, 1), jnp.float32)
        for c in range(NCHUNK):
            w_copies[c].wait()
            wb = wbuf[c % 2].astype(jnp.bfloat16)
            e = jnp.exp(jnp.dot(xb, wb, preferred_element_type=jnp.float32))
            s_mine = s_mine + jnp.sum(e, axis=1, keepdims=True)
            send_buf[c] = e.astype(jnp.bfloat16)
            if c + 2 < NCHUNK:
                w_copies.append(start_w(c + 2))
            rdma = pltpu.make_async_remote_copy(
                src_ref=send_buf.at[c],
                dst_ref=recv_buf.at[c],
                send_sem=send_sems.at[c],
                recv_sem=recv_sems.at[c],
                device_id=peer,
                device_id_type=pl.DeviceIdType.MESH,
            )
            if c < NCHUNK // 2:
                rdma.start()
            rdmas.append(rdma)

        sum_send[...] = s_mine
        sum_rdma = pltpu.make_async_remote_copy(
            src_ref=sum_send,
            dst_ref=sum_recv,
            send_sem=sum_sems.at[0],
            recv_sem=sum_sems.at[1],
            device_id=peer,
            device_id_type=pl.DeviceIdType.MESH,
        )
        sum_rdma.start()
        for c in range(NCHUNK // 2, NCHUNK):
            rdmas[c].start()
        sum_rdma.wait_recv()
        inv = 1.0 / (s_mine + sum_recv[...])

        out_copies = []
        def store_piece(idx, val, col_off):
            slot = idx % 2
            if idx >= 2:
                out_copies[idx - 2].wait()
            stage[slot] = val
            cp = pltpu.make_async_copy(
                stage.at[slot],
                out_ref.at[:, pl.ds(col_off, CC)],
                out_sems.at[slot],
            )
            cp.start()
            out_copies.append(cp)

        idx = 0
        for c in range(NCHUNK):
            store_piece(
                idx, send_buf[c].astype(jnp.float32) * inv, my_off + c * CC
            )
            idx += 1
        for c in range(NCHUNK):
            rdmas[c].wait_recv()
            store_piece(
                idx, recv_buf[c].astype(jnp.float32) * inv, peer_off + c * CC
            )
            idx += 1
        out_copies[-2].wait()
        out_copies[-1].wait()

        for c in range(NCHUNK):
            rdmas[c].wait_send()
        sum_rdma.wait_send()

    return pl.pallas_call(
        body,
        out_shape=jax.ShapeDtypeStruct((T, 2 * V_SHARD), jnp.float32),
        in_specs=[
            pl.BlockSpec(memory_space=pltpu.VMEM),
            pl.BlockSpec(memory_space=pltpu.HBM),
        ],
        out_specs=pl.BlockSpec(memory_space=pltpu.HBM),
        scratch_shapes=[
            pltpu.VMEM((2, D, CC), jnp.float32),
            pltpu.VMEM((NCHUNK, T, CC), jnp.bfloat16),
            pltpu.VMEM((NCHUNK, T, CC), jnp.bfloat16),
            pltpu.VMEM((2, T, CC), jnp.float32),
            pltpu.VMEM((T, 1), jnp.float32),
            pltpu.VMEM((T, 1), jnp.float32),
            pltpu.SemaphoreType.DMA((2,)),
            pltpu.SemaphoreType.DMA((NCHUNK,)),
            pltpu.SemaphoreType.DMA((NCHUNK,)),
            pltpu.SemaphoreType.DMA((2,)),
            pltpu.SemaphoreType.DMA((2,)),
        ],
        compiler_params=pltpu.CompilerParams(collective_id=0),
    )(x, W)
